# baseline (speedup 1.0000x reference)
import functools

import jax
import jax.numpy as jnp
from jax import lax
from jax.experimental import pallas as pl
from jax.experimental.pallas import tpu as pltpu
from jax.experimental.pallas import tpu_sc as plsc

N_NODES = 10000
N_EDGES = 320000
D_FEAT = 128
HID = 64
EXT = HID + 16

_NC = 2
_NS = 16
_NW = _NC * _NS

_CH = 80
_EPW = N_EDGES // _NW
_NCHUNK = _EPW // _CH
_NPAD = 10240
_RPT = _NPAD // _NS
_RCOPY = _CH

_BLK = 2000
_GRID = N_NODES // _BLK


def _dot_t(a, w):
    return lax.dot_general(a, w, (((1,), (1,)), ((), ())),
                           preferred_element_type=jnp.float32)


def _tc1_body(x_ref, wl_ref, wr_ref, e_ref, y_ref, z_ref, pk_ref):
    xb = x_ref[...]
    y_ref[...] = _dot_t(xb, wl_ref[...]).astype(jnp.bfloat16)
    z_ref[...] = _dot_t(xb, wr_ref[...])
    pk_ref[...] = jnp.left_shift(e_ref[0], 14) | e_ref[1]


def _tc1(x, Wl1, Wr1, edge_index):
    return pl.pallas_call(
        _tc1_body,
        out_shape=[
            jax.ShapeDtypeStruct((N_NODES, HID), jnp.bfloat16),
            jax.ShapeDtypeStruct((N_NODES, HID), jnp.float32),
            jax.ShapeDtypeStruct((N_EDGES,), jnp.int32),
        ],
    )(x, Wl1, Wr1, edge_index)


def _tc2_body(aggp_ref, cntp_ref, z1_ref, bl1_ref, wl2_ref, wr2_ref,
              y2_ref, z2_ref):
    a = (aggp_ref[0, :N_NODES, :] + aggp_ref[1, :N_NODES, :]).astype(
        jnp.float32)
    cv = cntp_ref[0, :N_NODES, :] + cntp_ref[1, :N_NODES, :]
    cnt = jnp.max(cv, axis=1, keepdims=True)
    recip = 1.0 / jnp.maximum(cnt, 1.0)
    h1 = jnp.maximum(a * recip + bl1_ref[...] + z1_ref[...], 0.0)
    y2_ref[...] = _dot_t(h1, wl2_ref[...]).astype(jnp.bfloat16)
    z2_ref[...] = jnp.concatenate(
        [_dot_t(h1, wr2_ref[...]),
         jnp.broadcast_to(recip, (N_NODES, EXT - HID))], axis=1)


def _tc2(aggp, cntp, z1, bl1, Wl2, Wr2):
    return pl.pallas_call(
        _tc2_body,
        out_shape=[
            jax.ShapeDtypeStruct((N_NODES, HID), jnp.bfloat16),
            jax.ShapeDtypeStruct((N_NODES, EXT), jnp.float32),
        ],
    )(aggp, cntp, z1, bl1, Wl2, Wr2)


def _tc3_body(aggp_ref, z2_ref, bl2_ref, wf1_ref, bf1_ref, wf2_ref, bf2_ref,
              out_ref):
    a = (aggp_ref[0, :N_NODES, :] + aggp_ref[1, :N_NODES, :]).astype(
        jnp.float32)
    recip = z2_ref[:, HID:HID + 1]
    h2 = jnp.maximum(a * recip + bl2_ref[...] + z2_ref[:, :HID], 0.0)
    g = jnp.sum(h2, axis=0, keepdims=True) * (1.0 / N_NODES)
    h = jnp.maximum(_dot_t(g, wf1_ref[...]) + bf1_ref[...], 0.0)
    o = _dot_t(h, wf2_ref[...]) + bf2_ref[...]
    m = jnp.max(o, axis=1, keepdims=True)
    e = jnp.exp(o - m)
    out_ref[...] = e / jnp.sum(e, axis=1, keepdims=True)


def _tc3(aggp, z2, bl2, Wf1, bf1, Wf2, bf2):
    nout = Wf2.shape[0]
    return pl.pallas_call(
        _tc3_body,
        out_shape=jax.ShapeDtypeStruct((1, nout), jnp.float32),
    )(aggp, z2, bl2, Wf1, bf1, Wf2, bf2)


_CNTW = 16


def _sc_agg(y, edges3d, d, stage_y, with_cnt=False):
    dt = y.dtype
    lw = 32 if dt == jnp.bfloat16 else 16
    mesh = plsc.VectorSubcoreMesh(core_axis_name="c", subcore_axis_name="s",
                                  num_cores=_NC, num_subcores=_NS)

    out_type = [jax.ShapeDtypeStruct((_NC, _NPAD, d), dt)]
    if with_cnt:
        out_type.append(
            jax.ShapeDtypeStruct((_NC, _NPAD, _CNTW), jnp.float32))

    @functools.partial(
        pl.kernel,
        out_type=out_type,
        mesh=mesh,
        scratch_types=[
            pltpu.VMEM((_NCHUNK, _CH), jnp.int32),
            [pltpu.VMEM((_CH,), jnp.int32)] * 3,
            [pltpu.VMEM((_CH,), jnp.int32)] * 3,
            [pltpu.VMEM((_CH, d), dt)] * 3,
            pltpu.VMEM((_CH, _CNTW), jnp.float32),
            pltpu.VMEM_SHARED((N_NODES if stage_y else 1, d), dt),
            pltpu.VMEM_SHARED((_NPAD, d), dt),
            pltpu.VMEM_SHARED((_NPAD if with_cnt else 1, _CNTW), jnp.float32),
            [pltpu.SemaphoreType.DMA] * 3,
            [pltpu.SemaphoreType.DMA] * 3,
        ],
        compiler_params=pltpu.CompilerParams(use_tc_tiling_on_sc=False),
    )
    def k(y_hbm, edges_hbm, *rest):
        if with_cnt:
            (out_hbm, cnt_hbm, pk_v, src_v, dst_v, rows_v, ones_v,
             y_sh, agg_sh, cnt_sh, gsem, ssem) = rest
        else:
            (out_hbm, pk_v, src_v, dst_v, rows_v, ones_v,
             y_sh, agg_sh, cnt_sh, gsem, ssem) = rest
        rows0_v = rows_v[0]
        c = lax.axis_index("c")
        s = lax.axis_index("s")
        wid = s * _NC + c

        if stage_y:
            @pl.when(s == 0)
            def _():
                pltpu.async_copy(y_hbm, y_sh, gsem[1])

        pltpu.async_copy(edges_hbm.at[wid], pk_v, gsem[0])

        def zrow(i, _):
            for jj in range(d // lw):
                rows0_v[i, pl.ds(jj * lw, lw)] = jnp.zeros((lw,), dt)
            return 0
        lax.fori_loop(0, _RCOPY, zrow, 0)
        for r in range(_RPT // _RCOPY):
            pltpu.sync_copy(
                rows0_v, agg_sh.at[pl.ds(s * _RPT + r * _RCOPY, _RCOPY)])

        if with_cnt:
            def czrow(i, val):
                ones_v[i, pl.ds(0, 16)] = jnp.full((16,), val, jnp.float32)
                return val
            lax.fori_loop(0, _CH, lambda i, v: czrow(i, 0.0), 0.0)
            for r in range(_RPT // _RCOPY):
                pltpu.sync_copy(
                    ones_v,
                    cnt_sh.at[pl.ds(s * _RPT + r * _RCOPY, _RCOPY)])
            lax.fori_loop(0, _CH, lambda i, v: czrow(i, 1.0), 1.0)

        pltpu.make_async_copy(edges_hbm.at[wid], pk_v, gsem[0]).wait()
        if stage_y:
            @pl.when(s == 0)
            def _():
                pltpu.make_async_copy(y_hbm, y_sh, gsem[1]).wait()

        plsc.subcore_barrier()

        y_src = y_sh if stage_y else y_hbm

        def ups(j, b):
            for jj in range(_CH // 16):
                p = pk_v[j, pl.ds(jj * 16, 16)]
                src_v[b][pl.ds(jj * 16, 16)] = lax.shift_right_logical(p, 14)

        def upd(j, b):
            for jj in range(_CH // 16):
                p = pk_v[j, pl.ds(jj * 16, 16)]
                dst_v[b][pl.ds(jj * 16, 16)] = lax.bitwise_and(p, 16383)

        def gi(j, b):
            pltpu.async_copy(y_src.at[src_v[b]], rows_v[b], gsem[b])

        def gw(b):
            pltpu.make_async_copy(
                y_src.at[src_v[b]], rows_v[b], gsem[b]).wait()

        def si(b):
            pltpu.async_copy(rows_v[b], agg_sh.at[dst_v[b]], ssem[b],
                             add=True)
            if with_cnt:
                pltpu.sync_copy(ones_v, cnt_sh.at[dst_v[b]], add=True)

        def sw(b):
            pltpu.make_async_copy(
                rows_v[b], agg_sh.at[dst_v[b]], ssem[b]).wait()

        ups(0, 0); upd(0, 0); gi(0, 0)
        gw(0); ups(1, 1); upd(1, 1); gi(1, 1); si(0)
        gw(1); ups(2, 2); upd(2, 2); gi(2, 2); si(1)

        def body(i, _):
            j = 3 * i + 2
            for k in range(3):
                b = (2 + k) % 3
                bn = (3 + k) % 3
                gw(b)
                sw(bn)
                ups(j + k + 1, bn)
                upd(j + k + 1, bn)
                gi(j + k + 1, bn)
                si(b)
            return 0
        lax.fori_loop(0, (_NCHUNK - 5) // 3, body, 0)

        gw(2); sw(0); ups(_NCHUNK - 2, 0); upd(_NCHUNK - 2, 0)
        gi(_NCHUNK - 2, 0); si(2)
        gw(0); sw(1); ups(_NCHUNK - 1, 1); upd(_NCHUNK - 1, 1)
        gi(_NCHUNK - 1, 1); si(0)
        gw(1); si(1)
        sw(2); sw(0); sw(1)

        plsc.subcore_barrier()

        for r in range(_RPT // _RCOPY):
            base = s * _RPT + r * _RCOPY
            pltpu.sync_copy(agg_sh.at[pl.ds(base, _RCOPY)], rows0_v)
            pltpu.sync_copy(rows0_v, out_hbm.at[c, pl.ds(base, _RCOPY)])
            if with_cnt:
                pltpu.sync_copy(cnt_sh.at[pl.ds(base, _RCOPY)], ones_v)
                pltpu.sync_copy(ones_v, cnt_hbm.at[c, pl.ds(base, _RCOPY)])

    res = k(y, edges3d)
    return res if with_cnt else res[0]


def kernel(x, edge_index, Wl1, bl1, Wr1, Wl2, bl2, Wr2, Wf1, bf1, Wf2, bf2):
    y1, z1, packed = _tc1(x, Wl1, Wr1, edge_index)
    edges3d = packed.reshape(_NW, _NCHUNK, _CH)
    aggp1, cntp1 = _sc_agg(y1, edges3d, HID, stage_y=True, with_cnt=True)
    y2, z2 = _tc2(aggp1, cntp1, z1, bl1.reshape(1, HID), Wl2, Wr2)
    aggp2 = _sc_agg(y2, edges3d, HID, stage_y=True)
    return _tc3(aggp2, z2, bl2.reshape(1, HID),
                Wf1, bf1.reshape(1, HID), Wf2, bf2.reshape(1, Wf2.shape[0]))
